# baseline (speedup 1.0000x reference)
import functools

import jax
import jax.numpy as jnp
from jax import lax
from jax.experimental import pallas as pl
from jax.experimental.pallas import tpu as pltpu
from jax.experimental.pallas import tpu_sc as plsc

NUM_FACTORS = 16
BATCH = 16384
L = 16
NC, NS = 2, 16
NW = NC * NS
BPW = BATCH // NW
CHUNK = 128
NCHUNK = BPW // CHUNK


def _build():
    mesh = plsc.VectorSubcoreMesh(core_axis_name="c", subcore_axis_name="s")

    @functools.partial(
        pl.kernel,
        mesh=mesh,
        compiler_params=pltpu.CompilerParams(
            needs_layout_passes=False, use_tc_tiling_on_sc=False),
        out_type=jax.ShapeDtypeStruct((BATCH,), jnp.float32),
        scratch_types=[
            pltpu.VMEM((NCHUNK, CHUNK), jnp.int32),
            pltpu.VMEM((NCHUNK, CHUNK), jnp.int32),
            pltpu.VMEM((BPW, NUM_FACTORS), jnp.float32),
            pltpu.VMEM((BPW, NUM_FACTORS), jnp.float32),
            pltpu.VMEM((BPW,), jnp.float32),
            pltpu.SemaphoreType.DMA,
        ],
    )
    def mf_forward(uid_hbm, eid_hbm, utab_hbm, etab_hbm, out_hbm,
                   uid_v, eid_v, u_v, e_v, o_v, sem):
        wid = lax.axis_index("s") * NC + lax.axis_index("c")
        base = wid * BPW
        row0 = wid * NCHUNK

        pltpu.sync_copy(uid_hbm.at[pl.ds(row0, NCHUNK)], uid_v)
        pltpu.sync_copy(eid_hbm.at[pl.ds(row0, NCHUNK)], eid_v)

        copies = []
        for j in range(NCHUNK):
            copies.append(pltpu.async_copy(
                utab_hbm.at[uid_v.at[j]], u_v.at[pl.ds(j * CHUNK, CHUNK)], sem))
            copies.append(pltpu.async_copy(
                etab_hbm.at[eid_v.at[j]], e_v.at[pl.ds(j * CHUNK, CHUNK)], sem))
        for cp in copies:
            cp.wait()

        lane = lax.iota(jnp.int32, L)

        def body(t, carry):
            acc = jnp.zeros((L,), jnp.float32)
            for k in range(L):
                r = t * L + k
                p = u_v[r, :] * e_v[r, :]
                acc = jnp.where(lane == k, jnp.sum(p), acc)
            o_v[pl.ds(t * L, L)] = acc
            return carry

        lax.fori_loop(0, BPW // L, body, 0)
        pltpu.sync_copy(o_v, out_hbm.at[pl.ds(base, BPW)])

    return mf_forward


_KERNEL = _build()


def kernel(user_id, event_id, user_table, event_table):
    uid2 = user_id.reshape(NW * NCHUNK, CHUNK)
    eid2 = event_id.reshape(NW * NCHUNK, CHUNK)
    return _KERNEL(uid2, eid2, user_table, event_table)
